# baseline (speedup 1.0000x reference)
import functools

import jax
import jax.numpy as jnp
from jax import lax
from jax.experimental import pallas as pl
from jax.experimental.pallas import tpu as pltpu
from jax.experimental.pallas import tpu_sc as plsc

NUM_CORES = 2
NUM_SUBCORES = 16
NUM_WORKERS = NUM_CORES * NUM_SUBCORES
LANES = 16
ILIST = 128
SUB = 2
GROUP = SUB * ILIST
NBUF = 2
FGROUP = 128
SLOT_START = 49


def _build(num_tokens, dim, pad_rows):
  tok_per_tile = num_tokens // NUM_WORKERS
  num_groups = tok_per_tile // GROUP
  num_fgroups = tok_per_tile // FGROUP
  assert tok_per_tile * NUM_WORKERS == num_tokens
  assert num_groups * GROUP == tok_per_tile
  assert num_groups % NBUF == 0 and num_groups // NBUF >= 2
  assert dim % LANES == 0

  mesh = plsc.VectorSubcoreMesh(
      core_axis_name="c", subcore_axis_name="s",
      num_cores=NUM_CORES, num_subcores=NUM_SUBCORES)

  @functools.partial(
      pl.kernel,
      out_type=jax.ShapeDtypeStruct((num_tokens + pad_rows, dim), jnp.float32),
      mesh=mesh,
      scratch_types=[
          pltpu.VMEM((tok_per_tile,), jnp.int32),
          pltpu.VMEM((NBUF, GROUP, dim), jnp.float32),
          pltpu.VMEM((NBUF, SUB, ILIST), jnp.int32),
          pltpu.VMEM((FGROUP,), jnp.int32),
          pltpu.VMEM((FGROUP,), jnp.int32),
          pltpu.VMEM((FGROUP, dim), jnp.float32),
          [pltpu.SemaphoreType.DMA] * NBUF,
          [pltpu.SemaphoreType.DMA] * NBUF,
      ],
      compiler_params=pltpu.CompilerParams(needs_layout_passes=False),
  )
  def gather_kernel(emb, slot, idx_hbm, out, idx_v, rows, idxs, sidx, spos,
                    frows, sem_g, sem_w):
    wid = lax.axis_index("s") * NUM_CORES + lax.axis_index("c")
    base = wid * tok_per_tile
    dump_row = num_tokens + wid
    lane = lax.broadcasted_iota(jnp.int32, (LANES,), 0)

    pltpu.sync_copy(idx_hbm.at[pl.ds(base, tok_per_tile)], idx_v)

    def start_gather(g, b):
      for s in range(SUB):
        for k in range(ILIST // LANES):
          idxs[b, s, pl.ds(k * LANES, LANES)] = idx_v[
              pl.ds(g * GROUP + s * ILIST + k * LANES, LANES)]
      for s in range(SUB):
        pltpu.async_copy(
            emb.at[idxs.at[b].at[s]],
            rows.at[b].at[pl.ds(s * ILIST, ILIST)], sem_g[b])

    def wait_gather(b):
      for s in range(SUB):
        pltpu.make_async_copy(
            emb.at[pl.ds(0, ILIST)],
            rows.at[b].at[pl.ds(s * ILIST, ILIST)], sem_g[b]).wait()

    def start_write(g, b):
      pltpu.async_copy(
          rows.at[b], out.at[pl.ds(base + g * GROUP, GROUP)], sem_w[b])

    def wait_write(b):
      pltpu.make_async_copy(
          rows.at[b], out.at[pl.ds(base, GROUP)], sem_w[b]).wait()

    for g in range(NBUF - 1):
      start_gather(g, g)

    wait_gather(0)
    start_write(0, 0)
    start_gather(NBUF - 1, NBUF - 1)

    def ring_block(k, first, last):
      def step(g, b):
        wait_gather(b)
        start_write(g, b)
        bp = (b - 1) % NBUF
        wait_write(bp)
        if not last or b == 0:
          start_gather(g + NBUF - 1, bp)

      for b in range(NBUF):
        g = k * NBUF + b
        if first and b == 0:
          continue
        if last and b > 0:
          wait_gather(b)
          start_write(g, b)
          wait_write(b - 1)
        else:
          step(g, b)

    ring_block(0, True, False)
    lax.fori_loop(
        1, num_groups // NBUF - 1,
        lambda k, _: (ring_block(k, False, False), ())[1], (), unroll=False)
    ring_block(num_groups // NBUF - 1, False, True)
    wait_write(NBUF - 1)

    def fixup_body(fg, _):
      loff = fg * FGROUP
      m_any = idx_v[pl.ds(loff, LANES)] <= SLOT_START
      for j in range(1, FGROUP // LANES):
        m_any = m_any | (idx_v[pl.ds(loff + j * LANES, LANES)] <= SLOT_START)

      @pl.when(jnp.any(m_any))
      def _fixup():
        for j in range(FGROUP // LANES):
          v = idx_v[pl.ds(loff + j * LANES, LANES)]
          m = v <= SLOT_START
          sidx[pl.ds(j * LANES, LANES)] = jnp.where(m, SLOT_START - v, 0)
          spos[pl.ds(j * LANES, LANES)] = jnp.where(
              m, base + loff + j * LANES + lane, dump_row)
        pltpu.sync_copy(slot.at[sidx], frows)
        pltpu.sync_copy(frows, out.at[spos])

      return ()

    lax.fori_loop(0, num_fgroups, fixup_body, (), unroll=False)

  return gather_kernel


def kernel(embeddings, slot_embeddings, _input):
  batch, seq = _input.shape
  dim = embeddings.shape[1]
  num_tokens = batch * seq
  pad_rows = NUM_WORKERS
  idx = _input.reshape(num_tokens)
  out = _build(num_tokens, dim, pad_rows)(embeddings, slot_embeddings, idx)
  return out[:num_tokens].reshape(batch, seq, dim)


# speedup vs baseline: 2.7279x; 2.7279x over previous
import functools

import jax
import jax.numpy as jnp
from jax import lax
from jax.experimental import pallas as pl
from jax.experimental.pallas import tpu as pltpu
from jax.experimental.pallas import tpu_sc as plsc

NUM_CORES = 2
NUM_SUBCORES = 16
NUM_WORKERS = NUM_CORES * NUM_SUBCORES
LANES = 16
ILIST = 128
SUB = 2
GROUP = SUB * ILIST
NBUF = 2
FGROUP = 128
SLOT_START = 49

SC_TOKENS = 212992
TB = 2048


def _sc_build(num_tokens, dim, pad_rows):
  tok_per_tile = num_tokens // NUM_WORKERS
  num_groups = tok_per_tile // GROUP
  num_fgroups = tok_per_tile // FGROUP
  assert tok_per_tile * NUM_WORKERS == num_tokens
  assert num_groups * GROUP == tok_per_tile
  assert num_fgroups * FGROUP == tok_per_tile
  assert num_groups % NBUF == 0 and num_groups // NBUF >= 2
  assert dim % LANES == 0

  mesh = plsc.VectorSubcoreMesh(
      core_axis_name="c", subcore_axis_name="s",
      num_cores=NUM_CORES, num_subcores=NUM_SUBCORES)

  @functools.partial(
      pl.kernel,
      out_type=jax.ShapeDtypeStruct((num_tokens + pad_rows, dim), jnp.float32),
      mesh=mesh,
      scratch_types=[
          pltpu.VMEM((tok_per_tile,), jnp.int32),
          pltpu.VMEM((NBUF, GROUP, dim), jnp.float32),
          pltpu.VMEM((NBUF, SUB, ILIST), jnp.int32),
          pltpu.VMEM((FGROUP,), jnp.int32),
          pltpu.VMEM((FGROUP,), jnp.int32),
          pltpu.VMEM((FGROUP, dim), jnp.float32),
          [pltpu.SemaphoreType.DMA] * NBUF,
          [pltpu.SemaphoreType.DMA] * NBUF,
      ],
      compiler_params=pltpu.CompilerParams(needs_layout_passes=False),
  )
  def gather_kernel(emb, slot, idx_hbm, out, idx_v, rows, idxs, sidx, spos,
                    frows, sem_g, sem_w):
    wid = lax.axis_index("s") * NUM_CORES + lax.axis_index("c")
    base = wid * tok_per_tile
    dump_row = num_tokens + wid
    lane = lax.broadcasted_iota(jnp.int32, (LANES,), 0)

    pltpu.sync_copy(idx_hbm.at[pl.ds(base, tok_per_tile)], idx_v)

    def start_gather(g, b):
      for s in range(SUB):
        for k in range(ILIST // LANES):
          idxs[b, s, pl.ds(k * LANES, LANES)] = idx_v[
              pl.ds(g * GROUP + s * ILIST + k * LANES, LANES)]
      for s in range(SUB):
        pltpu.async_copy(
            emb.at[idxs.at[b].at[s]],
            rows.at[b].at[pl.ds(s * ILIST, ILIST)], sem_g[b])

    def wait_gather(b):
      for s in range(SUB):
        pltpu.make_async_copy(
            emb.at[pl.ds(0, ILIST)],
            rows.at[b].at[pl.ds(s * ILIST, ILIST)], sem_g[b]).wait()

    def start_write(g, b):
      pltpu.async_copy(
          rows.at[b], out.at[pl.ds(base + g * GROUP, GROUP)], sem_w[b])

    def wait_write(b):
      pltpu.make_async_copy(
          rows.at[b], out.at[pl.ds(base, GROUP)], sem_w[b]).wait()

    for g in range(NBUF - 1):
      start_gather(g, g)

    wait_gather(0)
    start_write(0, 0)
    start_gather(NBUF - 1, NBUF - 1)

    def ring_block(k, first, last):
      def step(g, b):
        wait_gather(b)
        start_write(g, b)
        bp = (b - 1) % NBUF
        wait_write(bp)
        if not last or b == 0:
          start_gather(g + NBUF - 1, bp)

      for b in range(NBUF):
        g = k * NBUF + b
        if first and b == 0:
          continue
        if last and b > 0:
          wait_gather(b)
          start_write(g, b)
          wait_write(b - 1)
        else:
          step(g, b)

    ring_block(0, True, False)
    lax.fori_loop(
        1, num_groups // NBUF - 1,
        lambda k, _: (ring_block(k, False, False), ())[1], (), unroll=False)
    ring_block(num_groups // NBUF - 1, False, True)
    wait_write(NBUF - 1)

    def fixup_body(fg, _):
      loff = fg * FGROUP
      m_any = idx_v[pl.ds(loff, LANES)] <= SLOT_START
      for j in range(1, FGROUP // LANES):
        m_any = m_any | (idx_v[pl.ds(loff + j * LANES, LANES)] <= SLOT_START)

      @pl.when(jnp.any(m_any))
      def _fixup():
        for j in range(FGROUP // LANES):
          v = idx_v[pl.ds(loff + j * LANES, LANES)]
          m = v <= SLOT_START
          sidx[pl.ds(j * LANES, LANES)] = jnp.where(m, SLOT_START - v, 0)
          spos[pl.ds(j * LANES, LANES)] = jnp.where(
              m, base + loff + j * LANES + lane, dump_row)
        pltpu.sync_copy(slot.at[sidx], frows)
        pltpu.sync_copy(frows, out.at[spos])

      return ()

    lax.fori_loop(0, num_fgroups, fixup_body, (), unroll=False)

  return gather_kernel


def _tc_build(num_words, num_slots, dim, num_tokens, tc_start, tc_tokens):
  assert tc_tokens % TB == 0 and tc_start % TB == 0
  nblk = tc_tokens // TB
  blk0 = tc_start // TB

  def body(idx_ref, emb_ref, slot_ref, out_ref, table, sem):
    @pl.when(pl.program_id(0) == 0)
    def _init():
      pltpu.make_async_copy(emb_ref, table, sem).start()
      pltpu.make_async_copy(emb_ref, table, sem).wait()
      for k in range(num_slots):
        table[pl.ds(k, 1), :] = slot_ref[pl.ds(SLOT_START - k, 1), :]

    def row(i, _):
      s = idx_ref[0, 0, i]
      out_ref[pl.ds(i, 1), :] = table[pl.ds(s, 1), :]
      return ()

    lax.fori_loop(0, TB, row, (), unroll=16)

  return pl.pallas_call(
      body,
      grid=(nblk,),
      in_specs=[
          pl.BlockSpec((1, 1, TB), lambda i: (i, 0, 0),
                       memory_space=pltpu.SMEM),
          pl.BlockSpec(memory_space=pl.ANY),
          pl.BlockSpec((num_slots, dim), lambda i: (0, 0)),
      ],
      out_specs=pl.BlockSpec((TB, dim), lambda i: (blk0 + i, 0)),
      out_shape=jax.ShapeDtypeStruct((num_tokens, dim), jnp.float32),
      scratch_shapes=[
          pltpu.VMEM((num_words, dim), jnp.float32),
          pltpu.SemaphoreType.DMA,
      ],
      compiler_params=pltpu.CompilerParams(
          dimension_semantics=("arbitrary",),
          vmem_limit_bytes=100 * 1024 * 1024,
          disable_bounds_checks=True,
      ),
  )


def kernel(embeddings, slot_embeddings, _input):
  batch, seq = _input.shape
  num_words, dim = embeddings.shape
  num_slots = slot_embeddings.shape[0]
  num_tokens = batch * seq
  idx = _input.reshape(num_tokens)

  sc_tokens = SC_TOKENS
  tc_tokens = num_tokens - sc_tokens

  sc_out = _sc_build(sc_tokens, dim, NUM_WORKERS)(
      embeddings, slot_embeddings, idx[:sc_tokens])

  idx_tc = idx[sc_tokens:].reshape(tc_tokens // TB, 1, TB)
  full = _tc_build(num_words, num_slots, dim, num_tokens, sc_tokens,
                   tc_tokens)(idx_tc, embeddings, slot_embeddings)

  full = lax.dynamic_update_slice(full, sc_out[:sc_tokens], (0, 0))
  return full.reshape(batch, seq, dim)


# speedup vs baseline: 2.9330x; 1.0752x over previous
import functools

import jax
import jax.numpy as jnp
from jax import lax
from jax.experimental import pallas as pl
from jax.experimental.pallas import tpu as pltpu
from jax.experimental.pallas import tpu_sc as plsc

NUM_CORES = 2
NUM_SUBCORES = 16
NUM_WORKERS = NUM_CORES * NUM_SUBCORES
LANES = 16
ILIST = 128
SUB = 2
GROUP = SUB * ILIST
NBUF = 2
FGROUP = 128
SLOT_START = 49

SC_TOKENS = 196608
TB = 2048


def _sc_build(num_tokens, dim, pad_rows):
  tok_per_tile = num_tokens // NUM_WORKERS
  num_groups = tok_per_tile // GROUP
  num_fgroups = tok_per_tile // FGROUP
  assert tok_per_tile * NUM_WORKERS == num_tokens
  assert num_groups * GROUP == tok_per_tile
  assert num_fgroups * FGROUP == tok_per_tile
  assert num_groups % NBUF == 0 and num_groups // NBUF >= 2
  assert dim % LANES == 0

  mesh = plsc.VectorSubcoreMesh(
      core_axis_name="c", subcore_axis_name="s",
      num_cores=NUM_CORES, num_subcores=NUM_SUBCORES)

  @functools.partial(
      pl.kernel,
      out_type=jax.ShapeDtypeStruct((num_tokens + pad_rows, dim), jnp.float32),
      mesh=mesh,
      scratch_types=[
          pltpu.VMEM((tok_per_tile,), jnp.int32),
          pltpu.VMEM((NBUF, GROUP, dim), jnp.float32),
          pltpu.VMEM((NBUF, SUB, ILIST), jnp.int32),
          pltpu.VMEM((FGROUP,), jnp.int32),
          pltpu.VMEM((FGROUP,), jnp.int32),
          pltpu.VMEM((FGROUP, dim), jnp.float32),
          [pltpu.SemaphoreType.DMA] * NBUF,
          [pltpu.SemaphoreType.DMA] * NBUF,
      ],
      compiler_params=pltpu.CompilerParams(needs_layout_passes=False),
  )
  def gather_kernel(emb, slot, idx_hbm, out, idx_v, rows, idxs, sidx, spos,
                    frows, sem_g, sem_w):
    wid = lax.axis_index("s") * NUM_CORES + lax.axis_index("c")
    base = wid * tok_per_tile
    dump_row = num_tokens + wid
    lane = lax.broadcasted_iota(jnp.int32, (LANES,), 0)

    pltpu.sync_copy(idx_hbm.at[pl.ds(base, tok_per_tile)], idx_v)

    def start_gather(g, b):
      for s in range(SUB):
        for k in range(ILIST // LANES):
          idxs[b, s, pl.ds(k * LANES, LANES)] = idx_v[
              pl.ds(g * GROUP + s * ILIST + k * LANES, LANES)]
      for s in range(SUB):
        pltpu.async_copy(
            emb.at[idxs.at[b].at[s]],
            rows.at[b].at[pl.ds(s * ILIST, ILIST)], sem_g[b])

    def wait_gather(b):
      for s in range(SUB):
        pltpu.make_async_copy(
            emb.at[pl.ds(0, ILIST)],
            rows.at[b].at[pl.ds(s * ILIST, ILIST)], sem_g[b]).wait()

    def start_write(g, b):
      pltpu.async_copy(
          rows.at[b], out.at[pl.ds(base + g * GROUP, GROUP)], sem_w[b])

    def wait_write(b):
      pltpu.make_async_copy(
          rows.at[b], out.at[pl.ds(base, GROUP)], sem_w[b]).wait()

    for g in range(NBUF - 1):
      start_gather(g, g)

    wait_gather(0)
    start_write(0, 0)
    start_gather(NBUF - 1, NBUF - 1)

    def ring_block(k, first, last):
      def step(g, b):
        wait_gather(b)
        start_write(g, b)
        bp = (b - 1) % NBUF
        wait_write(bp)
        if not last or b == 0:
          start_gather(g + NBUF - 1, bp)

      for b in range(NBUF):
        g = k * NBUF + b
        if first and b == 0:
          continue
        if last and b > 0:
          wait_gather(b)
          start_write(g, b)
          wait_write(b - 1)
        else:
          step(g, b)

    ring_block(0, True, False)
    lax.fori_loop(
        1, num_groups // NBUF - 1,
        lambda k, _: (ring_block(k, False, False), ())[1], (), unroll=False)
    ring_block(num_groups // NBUF - 1, False, True)
    wait_write(NBUF - 1)

    def fixup_body(fg, _):
      loff = fg * FGROUP
      m_any = idx_v[pl.ds(loff, LANES)] <= SLOT_START
      for j in range(1, FGROUP // LANES):
        m_any = m_any | (idx_v[pl.ds(loff + j * LANES, LANES)] <= SLOT_START)

      @pl.when(jnp.any(m_any))
      def _fixup():
        for j in range(FGROUP // LANES):
          v = idx_v[pl.ds(loff + j * LANES, LANES)]
          m = v <= SLOT_START
          sidx[pl.ds(j * LANES, LANES)] = jnp.where(m, SLOT_START - v, 0)
          spos[pl.ds(j * LANES, LANES)] = jnp.where(
              m, base + loff + j * LANES + lane, dump_row)
        pltpu.sync_copy(slot.at[sidx], frows)
        pltpu.sync_copy(frows, out.at[spos])

      return ()

    lax.fori_loop(0, num_fgroups, fixup_body, (), unroll=False)

  return gather_kernel


def _tc_build(num_words, num_slots, dim, num_tokens, tc_start, tc_tokens):
  assert tc_tokens % TB == 0 and tc_start % TB == 0
  nblk = tc_tokens // TB
  blk0 = tc_start // TB

  def body(idx_ref, emb_ref, slot_ref, out_ref, table, sem):
    @pl.when(pl.program_id(0) == 0)
    def _init():
      pltpu.make_async_copy(emb_ref, table, sem).start()
      pltpu.make_async_copy(emb_ref, table, sem).wait()
      for k in range(num_slots):
        table[pl.ds(k, 1), :] = slot_ref[pl.ds(SLOT_START - k, 1), :]

    def row(i, _):
      s = idx_ref[0, 0, i]
      out_ref[pl.ds(i, 1), :] = table[pl.ds(s, 1), :]
      return ()

    lax.fori_loop(0, TB, row, (), unroll=32)

  return pl.pallas_call(
      body,
      grid=(nblk,),
      in_specs=[
          pl.BlockSpec((1, 1, TB), lambda i: (i, 0, 0),
                       memory_space=pltpu.SMEM),
          pl.BlockSpec(memory_space=pl.ANY),
          pl.BlockSpec((num_slots, dim), lambda i: (0, 0)),
      ],
      out_specs=pl.BlockSpec((TB, dim), lambda i: (blk0 + i, 0)),
      out_shape=jax.ShapeDtypeStruct((num_tokens, dim), jnp.float32),
      scratch_shapes=[
          pltpu.VMEM((num_words, dim), jnp.float32),
          pltpu.SemaphoreType.DMA,
      ],
      compiler_params=pltpu.CompilerParams(
          dimension_semantics=("arbitrary",),
          vmem_limit_bytes=100 * 1024 * 1024,
          disable_bounds_checks=True,
      ),
  )


def kernel(embeddings, slot_embeddings, _input):
  batch, seq = _input.shape
  num_words, dim = embeddings.shape
  num_slots = slot_embeddings.shape[0]
  num_tokens = batch * seq
  idx = _input.reshape(num_tokens)

  sc_tokens = SC_TOKENS
  tc_tokens = num_tokens - sc_tokens

  sc_out = _sc_build(sc_tokens, dim, NUM_WORKERS)(
      embeddings, slot_embeddings, idx[:sc_tokens])

  idx_tc = idx[sc_tokens:].reshape(tc_tokens // TB, 1, TB)
  full = _tc_build(num_words, num_slots, dim, num_tokens, sc_tokens,
                   tc_tokens)(idx_tc, embeddings, slot_embeddings)

  full = lax.dynamic_update_slice(full, sc_out[:sc_tokens], (0, 0))
  return full.reshape(batch, seq, dim)
